# baseline (speedup 1.0000x reference)
import functools

import jax
import jax.numpy as jnp
from jax import lax
from jax.experimental import pallas as pl
from jax.experimental.pallas import tpu as pltpu
from jax.experimental.pallas import tpu_sc as plsc

BATCH = 4096
SEQ_LEN = 200
VOCAB = 6
DIM = 64
TOTAL = BATCH * SEQ_LEN
PAIRS = TOTAL // 2

_info = plsc.get_sparse_core_info()
_NC, _NS = _info.num_cores, _info.num_subcores
_NW = _NC * _NS
_PER_W = PAIRS // _NW
_K = 128
_NG = _PER_W // _K
_NG2 = _NG // 2
_NV = DIM // 16


def _make_sc_gather():
    mesh = plsc.VectorSubcoreMesh(core_axis_name="c", subcore_axis_name="s")

    @functools.partial(
        pl.kernel,
        mesh=mesh,
        compiler_params=pltpu.CompilerParams(needs_layout_passes=False),
        out_type=jax.ShapeDtypeStruct((TOTAL, DIM), jnp.float32),
        scratch_types=[
            pltpu.VMEM((VOCAB, DIM), jnp.float32),
            pltpu.VMEM((VOCAB * VOCAB, 2 * DIM), jnp.float32),
            pltpu.VMEM_SHARED((VOCAB * VOCAB, 2 * DIM), jnp.float32),
            pltpu.VMEM((2 * _K,), jnp.int32),
            pltpu.VMEM((2 * _K,), jnp.int32),
            pltpu.VMEM((_K,), jnp.int32),
            pltpu.VMEM((_K,), jnp.int32),
            pltpu.VMEM((2 * _K, 2 * DIM), jnp.float32),
            pltpu.VMEM((2 * 2 * _K, DIM), jnp.float32),
        ] + [pltpu.SemaphoreType.DMA] * 6,
    )
    def gather_kernel(table_hbm, idx_hbm, out_hbm,
                      tab_v, ptab_v, ptab_sh, idx0_v, idx1_v, pidx0_v, pidx1_v,
                      rows_v, orows_v, *sems):
        isems = sems[0:2]
        gsems = sems[2:4]
        ssems = sems[4:6]
        wid = lax.axis_index("s") * _NC + lax.axis_index("c")
        wbase = wid * _PER_W

        pltpu.sync_copy(table_hbm, tab_v)

        tvregs = [[tab_v[a, pl.ds(k * 16, 16)] for k in range(_NV)]
                  for a in range(VOCAB)]
        for a in range(VOCAB):
            for b in range(VOCAB):
                p = a * VOCAB + b
                for k in range(_NV):
                    ptab_v[p, pl.ds(k * 16, 16)] = tvregs[a][k]
                    ptab_v[p, pl.ds(DIM + k * 16, 16)] = tvregs[b][k]

        @pl.when(lax.axis_index("s") == 0)
        def _():
            pltpu.sync_copy(ptab_v, ptab_sh)

        plsc.subcore_barrier()

        lanes2 = lax.iota(jnp.int32, 16) * 2
        idxbufs = [idx0_v, idx1_v]
        pidxbufs = [pidx0_v, pidx1_v]
        rbufs = [rows_v.at[pl.ds(b * _K, _K)] for b in range(2)]
        obufs = [orows_v.at[pl.ds(b * 2 * _K, 2 * _K)] for b in range(2)]

        def idx_src(g):
            return idx_hbm.at[pl.ds(2 * (wbase + g * _K), 2 * _K)]

        def out_dst(g):
            return out_hbm.at[pl.ds(2 * (wbase + g * _K), 2 * _K)]

        def start_idx(g, b):
            return pltpu.async_copy(idx_src(g), idxbufs[b], isems[b])

        def encode(b):
            for i in range(_K // 16):
                av = plsc.load_gather(idxbufs[b], [lanes2 + i * 32])
                bv = plsc.load_gather(idxbufs[b], [lanes2 + (i * 32 + 1)])
                pidxbufs[b][pl.ds(i * 16, 16)] = av * VOCAB + bv

        def repack(b):
            rbase = b * _K
            obase = b * 2 * _K

            def body(r, _):
                row = rbase + 2 * r
                orow = obase + 4 * r
                for u in range(2):
                    for k in range(_NV):
                        orows_v[orow + 2 * u, pl.ds(k * 16, 16)] = (
                            rows_v[row + u, pl.ds(k * 16, 16)])
                    for k in range(_NV):
                        orows_v[orow + 2 * u + 1, pl.ds(k * 16, 16)] = (
                            rows_v[row + u, pl.ds(DIM + k * 16, 16)])
                return 0

            lax.fori_loop(0, _K // 2, body, 0)

        start_idx(0, 0)
        start_idx(1, 1)

        def step(g2, _):
            for b in range(2):
                g = 2 * g2 + b
                pltpu.make_async_copy(idx_src(g), idxbufs[b], isems[b]).wait()
                encode(b)
                pltpu.async_copy(ptab_sh.at[pidxbufs[b]], rbufs[b], gsems[b])

                @pl.when(g2 < _NG2 - 1)
                def _():
                    start_idx(g + 2, b)

                bp = 1 - b
                gp = g - 1
                sthresh = 1 if b else 2

                def drain():
                    pltpu.make_async_copy(ptab_sh.at[pidxbufs[bp]],
                                          rbufs[bp], gsems[bp]).wait()

                    @pl.when(g2 >= sthresh)
                    def _():
                        pltpu.make_async_copy(obufs[bp], out_dst(gp),
                                              ssems[bp]).wait()

                    repack(bp)
                    pltpu.async_copy(obufs[bp], out_dst(gp), ssems[bp])

                if b == 0:
                    @pl.when(g2 > 0)
                    def _():
                        drain()
                else:
                    drain()
            return 0

        lax.fori_loop(0, _NG2, step, 0)

        glast = _NG - 1
        pltpu.make_async_copy(ptab_sh.at[pidxbufs[1]], rbufs[1],
                              gsems[1]).wait()
        pltpu.make_async_copy(obufs[1], out_dst(glast), ssems[1]).wait()
        repack(1)
        pltpu.async_copy(obufs[1], out_dst(glast), ssems[1])
        pltpu.make_async_copy(obufs[0], out_dst(glast - 1), ssems[0]).wait()
        pltpu.make_async_copy(obufs[1], out_dst(glast), ssems[1]).wait()

    return gather_kernel


_sc_gather = _make_sc_gather()


def kernel(indices, table):
    flat_idx = indices.reshape(-1).astype(jnp.int32)
    return _sc_gather(table, flat_idx)


# speedup vs baseline: 4.9052x; 4.9052x over previous
import functools

import jax
import jax.numpy as jnp
from jax import lax
from jax.experimental import pallas as pl
from jax.experimental.pallas import tpu as pltpu
from jax.experimental.pallas import tpu_sc as plsc

BATCH = 4096
SEQ_LEN = 200
VOCAB = 6
DIM = 64
TOTAL = BATCH * SEQ_LEN

_info = plsc.get_sparse_core_info()
_NC, _NS = _info.num_cores, _info.num_subcores
_NW = _NC * _NS
_PER_W = TOTAL // _NW
_BPW = _PER_W // SEQ_LEN
_CH = 512
_NCH = _PER_W // _CH

_DNUMS = lax.GatherDimensionNumbers(
    offset_dims=(), collapsed_slice_dims=(0,), start_index_map=(0,))


def _dgather(src, idx):
    return lax.gather(src, idx[:, None], _DNUMS, slice_sizes=(1,),
                      mode=lax.GatherScatterMode.PROMISE_IN_BOUNDS)


def _make_sc_gather():
    mesh = plsc.VectorSubcoreMesh(core_axis_name="c", subcore_axis_name="s")

    @functools.partial(
        pl.kernel,
        mesh=mesh,
        compiler_params=pltpu.CompilerParams(needs_layout_passes=False),
        out_type=jax.ShapeDtypeStruct((DIM, TOTAL), jnp.float32),
        scratch_types=[
            pltpu.VMEM((VOCAB, DIM), jnp.float32),
            pltpu.VMEM((DIM, 16), jnp.float32),
            pltpu.VMEM((SEQ_LEN, _BPW), jnp.int32),
            pltpu.VMEM((DIM, _CH), jnp.float32),
            pltpu.VMEM((DIM, _CH), jnp.float32),
        ] + [pltpu.SemaphoreType.DMA] * 2,
    )
    def gather_kernel(table_hbm, idx_t_hbm, out_hbm,
                      tab_v, ttab_v, idx_v, b0_v, b1_v, *ssems):
        wid = lax.axis_index("s") * _NC + lax.axis_index("c")
        wbase = wid * _PER_W

        pltpu.sync_copy(table_hbm, tab_v)
        pltpu.sync_copy(idx_t_hbm.at[:, pl.ds(wid * _BPW, _BPW)], idx_v)

        rows = jnp.minimum(lax.iota(jnp.int32, 16), VOCAB - 1)
        for d in range(DIM):
            cols = jnp.full((16,), d, jnp.int32)
            ttab_v[d, :] = plsc.load_gather(tab_v, [rows, cols])

        blocks = [b0_v, b1_v]
        lanes = lax.iota(jnp.int32, 16)

        def out_dst(c):
            return out_hbm.at[:, pl.ds(wbase + c * _CH, _CH)]

        def compute(c, b):
            ob = blocks[b]

            @plsc.parallel_loop(0, _CH // 16, unroll=2)
            def _(j):
                p = c * _CH + j * 16 + lanes
                idxv = plsc.load_gather(idx_v, [p % SEQ_LEN, p // SEQ_LEN])
                vals = [_dgather(ttab_v[d, :], idxv) for d in range(DIM)]
                for d in range(DIM):
                    ob[d, pl.ds(j * 16, 16)] = vals[d]

        def step(c2, _):
            for b in range(2):
                c = 2 * c2 + b

                @pl.when(c2 >= 1)
                def _():
                    pltpu.make_async_copy(blocks[b], out_dst(c),
                                          ssems[b]).wait()

                compute(c, b)
                pltpu.async_copy(blocks[b], out_dst(c), ssems[b])
            return 0

        lax.fori_loop(0, _NCH // 2, step, 0)
        pltpu.make_async_copy(blocks[0], out_dst(0), ssems[0]).wait()
        pltpu.make_async_copy(blocks[1], out_dst(1), ssems[1]).wait()

    return gather_kernel


_sc_gather = _make_sc_gather()


def kernel(indices, table):
    idx_t = indices.T.astype(jnp.int32)
    out_t = _sc_gather(table, idx_t)
    return out_t.T


# speedup vs baseline: 5.0848x; 1.0366x over previous
import functools

import jax
import jax.numpy as jnp
from jax import lax
from jax.experimental import pallas as pl
from jax.experimental.pallas import tpu as pltpu
from jax.experimental.pallas import tpu_sc as plsc

BATCH = 4096
SEQ_LEN = 200
VOCAB = 6
DIM = 64
TOTAL = BATCH * SEQ_LEN

_info = plsc.get_sparse_core_info()
_NC, _NS = _info.num_cores, _info.num_subcores
_NW = _NC * _NS
_PER_W = TOTAL // _NW
_BPW = _PER_W // SEQ_LEN
_CH = 512
_NCH = _PER_W // _CH

_DNUMS = lax.GatherDimensionNumbers(
    offset_dims=(), collapsed_slice_dims=(0,), start_index_map=(0,))


def _dgather(src, idx):
    return lax.gather(src, idx[:, None], _DNUMS, slice_sizes=(1,),
                      mode=lax.GatherScatterMode.PROMISE_IN_BOUNDS)


def _make_sc_gather():
    mesh = plsc.VectorSubcoreMesh(core_axis_name="c", subcore_axis_name="s")

    @functools.partial(
        pl.kernel,
        mesh=mesh,
        compiler_params=pltpu.CompilerParams(needs_layout_passes=False),
        out_type=jax.ShapeDtypeStruct((DIM, TOTAL), jnp.float32),
        scratch_types=[
            pltpu.VMEM((VOCAB, DIM), jnp.float32),
            pltpu.VMEM((DIM, 16), jnp.float32),
            pltpu.VMEM((SEQ_LEN, _BPW), jnp.int32),
            pltpu.VMEM((DIM, _CH), jnp.float32),
            pltpu.VMEM((DIM, _CH), jnp.float32),
        ] + [pltpu.SemaphoreType.DMA] * 2,
    )
    def gather_kernel(table_hbm, idx_t_hbm, out_hbm,
                      tab_v, ttab_v, idx_v, b0_v, b1_v, *ssems):
        wid = lax.axis_index("s") * _NC + lax.axis_index("c")
        wbase = wid * _PER_W

        pltpu.sync_copy(table_hbm, tab_v)
        pltpu.sync_copy(idx_t_hbm.at[:, pl.ds(wid * _BPW, _BPW)], idx_v)

        rows = jnp.minimum(lax.iota(jnp.int32, 16), VOCAB - 1)
        for d in range(DIM):
            cols = jnp.full((16,), d, jnp.int32)
            ttab_v[d, :] = plsc.load_gather(tab_v, [rows, cols])

        blocks = [b0_v, b1_v]
        lanes = lax.iota(jnp.int32, 16)

        def out_dst(c):
            return out_hbm.at[:, pl.ds(wbase + c * _CH, _CH)]

        def compute(c, b):
            ob = blocks[b]

            @plsc.parallel_loop(0, _CH // 16, unroll=2)
            def _(j):
                p0 = c * _CH + j * 16
                t0 = p0 % SEQ_LEN
                b0 = p0 // SEQ_LEN
                tv = t0 + lanes
                wrap = tv >= SEQ_LEN
                tv = jnp.where(wrap, tv - SEQ_LEN, tv)
                bv = jnp.where(wrap, b0 + 1, b0)
                idxv = plsc.load_gather(idx_v, [tv, bv])
                vals = [_dgather(ttab_v[d, :], idxv) for d in range(DIM)]
                for d in range(DIM):
                    ob[d, pl.ds(j * 16, 16)] = vals[d]

        def step(c2, _):
            for b in range(2):
                c = 2 * c2 + b

                @pl.when(c2 >= 1)
                def _():
                    pltpu.make_async_copy(blocks[b], out_dst(c),
                                          ssems[b]).wait()

                compute(c, b)
                pltpu.async_copy(blocks[b], out_dst(c), ssems[b])
            return 0

        lax.fori_loop(0, _NCH // 2, step, 0)
        pltpu.make_async_copy(blocks[0], out_dst(0), ssems[0]).wait()
        pltpu.make_async_copy(blocks[1], out_dst(1), ssems[1]).wait()

    return gather_kernel


_sc_gather = _make_sc_gather()


def kernel(indices, table):
    idx_t = indices.T.astype(jnp.int32)
    out_t = _sc_gather(table, idx_t)
    return out_t.T
